# baseline (speedup 1.0000x reference)
import jax
import jax.numpy as jnp
from jax import lax
from jax.experimental import pallas as pl
from jax.experimental.pallas import tpu as pltpu


_B, _N, _IN, _HID, _OUT = 4, 10000, 128, 256, 128


def _leaky(x):
    return jnp.where(x >= 0, x, 0.3 * x)


def _dot(a, b):
    return jnp.dot(a, b, preferred_element_type=jnp.float32)


def _body(ns_ref, nid_ref, emb_hbm, w_hbm, q_hbm, qb_hbm,
          wk_hbm, wb_hbm, o_ref, erows_v, crow_v, wrows_v,
          q_v, qb_v, wk_v, wb_v,
          sem_e, sem_c, sem_w, sem_q, sem_qb, sem_wk, sem_wb):
    nid = nid_ref[0]
    col0 = pl.multiple_of(nid & -128, 128)
    lane = nid - col0
    b0 = 2 * pl.program_id(0)

    for i in range(64):
        pltpu.make_async_copy(
            emb_hbm.at[pl.ds(b0, 2), pl.ds(ns_ref[i], 1), :],
            erows_v.at[:, pl.ds(i, 1)], sem_e,
        ).start()
    cp_q = pltpu.make_async_copy(q_hbm, q_v, sem_q)
    cp_qb = pltpu.make_async_copy(qb_hbm, qb_v, sem_qb)
    cp_q.start(); cp_qb.start()
    for i in range(64):
        pltpu.make_async_copy(
            w_hbm.at[pl.ds(ns_ref[i], 1), pl.ds(col0, 128)],
            wrows_v.at[pl.ds(i, 1)], sem_w,
        ).start()
    pltpu.make_async_copy(
        emb_hbm.at[pl.ds(b0, 2), pl.ds(nid, 1), :], crow_v, sem_c,
    ).start()
    cp_wk = pltpu.make_async_copy(wk_hbm, wk_v, sem_wk)
    cp_wb = pltpu.make_async_copy(wb_hbm, wb_v, sem_wb)
    cp_wk.start(); cp_wb.start()

    for i in range(64):
        pltpu.make_async_copy(
            emb_hbm.at[pl.ds(0, 2), pl.ds(0, 1), :],
            erows_v.at[:, pl.ds(i, 1)], sem_e,
        ).wait()
    cp_q.wait()
    cp_qb.wait()
    q = q_v[:]
    qb = qb_v[:]

    ne_rows = erows_v[:].reshape(2 * 64, _IN)
    h_all = _leaky(_dot(ne_rows, q) + qb[None, :])

    for i in range(64):
        pltpu.make_async_copy(
            w_hbm.at[pl.ds(0, 1), pl.ds(col0, 128)],
            wrows_v.at[pl.ds(i, 1)], sem_w,
        ).wait()
    lsel = (lax.broadcasted_iota(jnp.int32, (128, 1), 0) == lane
            ).astype(jnp.float32)
    w64 = _dot(wrows_v[:], lsel)
    denom = jnp.sum(w64) + 1e-6

    pltpu.make_async_copy(
        emb_hbm.at[pl.ds(0, 2), pl.ds(0, 1), :], crow_v, sem_c,
    ).wait()
    ce = crow_v[:].reshape(2, _IN)

    wt = jnp.concatenate([w64.reshape(1, 64)] * 2, axis=1)
    bsel = (lax.broadcasted_iota(jnp.int32, (2, 128), 1) // 64
            == lax.broadcasted_iota(jnp.int32, (2, 128), 0))
    w3 = jnp.where(bsel, wt, 0.0)
    wsm = _dot(w3, h_all) / denom
    cc = jnp.concatenate([ce, wsm], axis=1)
    cp_wk.wait()
    cp_wb.wait()
    h2 = _leaky(_dot(cc, wk_v[:]) + wb_v[:][None, :])
    nrm = jnp.sqrt(jnp.sum(h2 * h2, axis=1, keepdims=True)) + 1e-6
    o_ref[0] = jnp.concatenate(
        [h2 / nrm, jnp.zeros((6, _OUT), jnp.float32)], axis=0)


def kernel(embeddings, weights, Q_kernel, Q_bias, W_kernel, W_bias,
           neighbor_set, node_id):
    ns = neighbor_set.astype(jnp.int32)
    nid1 = jnp.asarray(node_id, jnp.int32).reshape(1)
    hbm = pl.BlockSpec(memory_space=pltpu.MemorySpace.HBM)
    smem = pl.BlockSpec(memory_space=pltpu.MemorySpace.SMEM)
    res = pl.pallas_call(
        _body,
        grid=(2,),
        in_specs=[smem, smem, hbm, hbm, hbm, hbm, hbm, hbm],
        out_specs=pl.BlockSpec((1, 8, _OUT), lambda c: (c, 0, 0)),
        out_shape=jax.ShapeDtypeStruct((2, 8, _OUT), jnp.float32),
        compiler_params=pltpu.CompilerParams(
            dimension_semantics=("parallel",),
        ),
        scratch_shapes=[
            pltpu.VMEM((2, 64, _IN), jnp.float32),
            pltpu.VMEM((2, 1, _IN), jnp.float32),
            pltpu.VMEM((64, 128), jnp.float32),
            pltpu.VMEM((_IN, _HID), jnp.float32),
            pltpu.VMEM((_HID,), jnp.float32),
            pltpu.VMEM((_IN + _HID, _OUT), jnp.float32),
            pltpu.VMEM((_OUT,), jnp.float32),
            pltpu.SemaphoreType.DMA,
            pltpu.SemaphoreType.DMA,
            pltpu.SemaphoreType.DMA,
            pltpu.SemaphoreType.DMA,
            pltpu.SemaphoreType.DMA,
            pltpu.SemaphoreType.DMA,
            pltpu.SemaphoreType.DMA,
        ],
    )(ns, nid1, embeddings, weights, Q_kernel, Q_bias, W_kernel, W_bias)
    return res[:, :2, :].reshape(_B, _OUT)


# speedup vs baseline: 1.8883x; 1.8883x over previous
import jax
import jax.numpy as jnp
from jax import lax
from jax.experimental import pallas as pl
from jax.experimental.pallas import tpu as pltpu


_B, _N, _IN, _HID, _OUT = 4, 10000, 128, 256, 128


def _leaky(x):
    return jnp.where(x >= 0, x, 0.3 * x)


def _dot(a, b):
    return jnp.dot(a, b, preferred_element_type=jnp.float32)


def _body(ns_ref, nid_ref, emb_hbm, w_hbm, q_hbm, qb_hbm,
          wk_hbm, wb_hbm, o_ref, erows_v, crow_v, wrows_v,
          q_v, qb_v, wk_v, wb_v,
          sem_e, sem_c, sem_w, sem_q, sem_qb, sem_wk, sem_wb):
    nid = nid_ref[0]
    col0 = pl.multiple_of(nid & -128, 128)
    lane = nid - col0

    for i in range(64):
        pltpu.make_async_copy(
            emb_hbm.at[:, pl.ds(ns_ref[i], 1), :], erows_v.at[:, pl.ds(i, 1)],
            sem_e,
        ).start()
    cp_q = pltpu.make_async_copy(q_hbm, q_v, sem_q)
    cp_qb = pltpu.make_async_copy(qb_hbm, qb_v, sem_qb)
    cp_q.start(); cp_qb.start()
    for i in range(64):
        pltpu.make_async_copy(
            w_hbm.at[pl.ds(ns_ref[i], 1), pl.ds(col0, 128)],
            wrows_v.at[pl.ds(i, 1)], sem_w,
        ).start()
    pltpu.make_async_copy(
        emb_hbm.at[:, pl.ds(nid, 1), :], crow_v, sem_c,
    ).start()
    cp_wk = pltpu.make_async_copy(wk_hbm, wk_v, sem_wk)
    cp_wb = pltpu.make_async_copy(wb_hbm, wb_v, sem_wb)
    cp_wk.start(); cp_wb.start()

    for i in range(64):
        pltpu.make_async_copy(
            emb_hbm.at[:, pl.ds(0, 1), :], erows_v.at[:, pl.ds(i, 1)], sem_e,
        ).wait()
    cp_q.wait()
    cp_qb.wait()
    q = q_v[:]
    qb = qb_v[:]

    ne_rows = erows_v[:].reshape(_B * 64, _IN)
    h_all = _leaky(_dot(ne_rows, q) + qb[None, :])

    for i in range(64):
        pltpu.make_async_copy(
            w_hbm.at[pl.ds(0, 1), pl.ds(col0, 128)],
            wrows_v.at[pl.ds(i, 1)], sem_w,
        ).wait()
    lsel = (lax.broadcasted_iota(jnp.int32, (128, 1), 0) == lane
            ).astype(jnp.float32)
    w64 = _dot(wrows_v[:], lsel)
    denom = jnp.sum(w64) + 1e-6

    pltpu.make_async_copy(
        emb_hbm.at[:, pl.ds(0, 1), :], crow_v, sem_c,
    ).wait()
    ce = crow_v[:].reshape(_B, _IN)

    wt = jnp.concatenate([w64.reshape(1, 64)] * _B, axis=1)
    bsel = (lax.broadcasted_iota(jnp.int32, (_B, _B * 64), 1) // 64
            == lax.broadcasted_iota(jnp.int32, (_B, _B * 64), 0))
    w3 = jnp.where(bsel, wt, 0.0)
    wsm = _dot(w3, h_all) / denom
    cc = jnp.concatenate([ce, wsm], axis=1)
    cp_wk.wait()
    cp_wb.wait()
    h2 = _leaky(_dot(cc, wk_v[:]) + wb_v[:][None, :])
    nrm = jnp.sqrt(jnp.sum(h2 * h2, axis=1, keepdims=True)) + 1e-6
    o_ref[:] = h2 / nrm


def kernel(embeddings, weights, Q_kernel, Q_bias, W_kernel, W_bias,
           neighbor_set, node_id):
    ns = neighbor_set.astype(jnp.int32)
    nid1 = jnp.asarray(node_id, jnp.int32).reshape(1)
    vmem = pl.BlockSpec(memory_space=pltpu.MemorySpace.VMEM)
    hbm = pl.BlockSpec(memory_space=pltpu.MemorySpace.HBM)
    smem = pl.BlockSpec(memory_space=pltpu.MemorySpace.SMEM)
    return pl.pallas_call(
        _body,
        in_specs=[smem, smem, hbm, hbm, hbm, hbm, hbm, hbm],
        out_specs=vmem,
        out_shape=jax.ShapeDtypeStruct((_B, _OUT), jnp.float32),
        scratch_shapes=[
            pltpu.VMEM((_B, 64, _IN), jnp.float32),
            pltpu.VMEM((_B, 1, _IN), jnp.float32),
            pltpu.VMEM((64, 128), jnp.float32),
            pltpu.VMEM((_IN, _HID), jnp.float32),
            pltpu.VMEM((_HID,), jnp.float32),
            pltpu.VMEM((_IN + _HID, _OUT), jnp.float32),
            pltpu.VMEM((_OUT,), jnp.float32),
            pltpu.SemaphoreType.DMA,
            pltpu.SemaphoreType.DMA,
            pltpu.SemaphoreType.DMA,
            pltpu.SemaphoreType.DMA,
            pltpu.SemaphoreType.DMA,
            pltpu.SemaphoreType.DMA,
            pltpu.SemaphoreType.DMA,
        ],
    )(ns, nid1, embeddings, weights, Q_kernel, Q_bias, W_kernel, W_bias)
